# baseline (speedup 1.0000x reference)
import functools

import jax
import jax.numpy as jnp
from jax import lax
from jax.experimental import pallas as pl
from jax.experimental.pallas import tpu as pltpu

_G = 512
_H = 8192
_SEQ = 200
_K = 4096
_CB = 1024
_NC = _H // _CB

_dot = functools.partial(jnp.dot, preferred_element_type=jnp.float32)


def _keys_of(raw):
    return lax.bitcast_convert_type(raw, jnp.uint32)


def _vsteps(keys, t, g0, n):
    for j in range(n):
        b = (jnp.uint32(30) - (g0 + j).astype(jnp.uint32))
        cand = t | (jnp.uint32(1) << b)
        cnt = jnp.sum((keys >= cand).astype(jnp.int32))
        t = lax.select(cnt >= _K, cand, t)
    return t


def _tie_break(keys, t):
    cgt = jnp.sum((keys > t).astype(jnp.int32))
    r = _K - cgt
    eq = keys == t
    idx = lax.broadcasted_iota(jnp.int32, (1, _H), 1)

    def istep(i, q):
        b = 12 - i
        cand = q | (jnp.int32(1) << b)
        f = jnp.sum((eq & (idx < cand)).astype(jnp.int32))
        return lax.select(f < r, cand, q)

    q = lax.fori_loop(0, 13, istep, jnp.int32(0))
    member = (keys > t) | (eq & (idx <= q))
    return jnp.where(member & (keys > jnp.uint32(0)),
                     jnp.float32(1.0), jnp.float32(0.0))


def _topk_binary(raw):
    keys = _keys_of(raw)
    t = _vsteps(keys, jnp.uint32(0), jnp.int32(0), 31)
    return _tie_break(keys, t)


def _body(x_ref, ei_ref, wihT_ref, whhT_ref, bl_ref,
          w0_ref, b0_ref, w1_ref, b1_ref,
          wg0_hbm, wc1_hbm, wg_ref, wc_ref, bg_ref, bc_ref,
          mask_ref, bin_ref,
          xw_ref, emb_ref, act_ref, y_ref, raw_ref, acc_ref,
          wg0_v, wc1_v, t_ref, sem_g, sem_c):
    s = pl.program_id(0)
    p = s // _NC
    c = s % _NC

    @pl.when(s == 0)
    def _prefetch_and_lstm():
        pltpu.make_async_copy(wg0_hbm, wg0_v, sem_g).start()
        pltpu.make_async_copy(wc1_hbm, wc1_v, sem_c).start()

        xw_ref[...] = _dot(x_ref[...], wihT_ref[...])

        def step(tt, hc):
            h, cc = hc
            gates = xw_ref[pl.ds(tt, 1), :] + _dot(h, whhT_ref[...]) + bl_ref[...]
            ig = jax.nn.sigmoid(gates[:, 0:_G])
            fg = jax.nn.sigmoid(gates[:, _G:2 * _G])
            gg = jnp.tanh(gates[:, 2 * _G:3 * _G])
            og = jax.nn.sigmoid(gates[:, 3 * _G:4 * _G])
            cc = fg * cc + ig * gg
            h = og * jnp.tanh(cc)
            return (h, cc)

        z = jnp.zeros((1, _G), jnp.float32)
        h, _ = lax.fori_loop(0, _SEQ, step, (z, z))

        emb = jax.nn.relu(_dot(ei_ref[...], w0_ref[...]) + b0_ref[...])
        emb = _dot(emb, w1_ref[...]) + b1_ref[...]
        embedding = emb * h
        emb_ref[...] = embedding
        act_ref[...] = jax.nn.relu(embedding)
        acc_ref[...] = jnp.zeros((1, _G), jnp.float32)
        pltpu.make_async_copy(wg0_hbm, wg0_v, sem_g).wait()

    @pl.when((s == _NC) & (c == 0))
    def _wait_wc1():
        pltpu.make_async_copy(wc1_hbm, wc1_v, sem_c).wait()

    @pl.when(p % 2 == 0)
    def _wg_phase():
        off = pl.multiple_of(c * _CB, _CB)

        def compute(wgm):
            y_c = _dot(act_ref[...], wgm) + bg_ref[0, pl.ds(0, 1), pl.ds(off, _CB)]
            y_ref[pl.ds(0, 1), pl.ds(off, _CB)] = y_c

        @pl.when(p == 0)
        def _use_prefetched():
            compute(wg0_v[:, pl.ds(off, _CB)])

        @pl.when(p > 0)
        def _use_stream():
            compute(wg_ref[...])

        @pl.when(c == _NC - 1)
        def _finish_layer():
            y = y_ref[...]
            mn = jnp.min(y)
            mx = jnp.max(y)
            raw = (y - mn) / (mx - mn)
            raw_ref[...] = raw
            mask_ref[...] = raw.reshape(1, 1, _H)

            @pl.when(p == 6)
            def _last_binary():
                bin_ref[...] = _topk_binary(raw).reshape(1, 1, _H)

    @pl.when(p % 2 == 1)
    def _wc_phase():
        off = pl.multiple_of(c * _CB, _CB)
        raw_c = raw_ref[pl.ds(0, 1), pl.ds(off, _CB)]

        @pl.when(p == 1)
        def _use_prefetched():
            acc_ref[...] += _dot(raw_c, wc1_v[pl.ds(off, _CB), :])

        @pl.when(p > 1)
        def _use_stream():
            acc_ref[...] += _dot(raw_c, wc_ref[...])

        keys = _keys_of(raw_ref[...])

        @pl.when(c == 0)
        def _tk_head():
            t_ref[0] = _vsteps(keys, jnp.uint32(0), jnp.int32(0), 5)

        @pl.when((c >= 1) & (c <= 5))
        def _tk_mid():
            t_ref[0] = _vsteps(keys, t_ref[0], c * 5, 5)

        @pl.when(c == 6)
        def _tk_last():
            t_ref[0] = _vsteps(keys, t_ref[0], jnp.int32(30), 1)

        @pl.when(c == _NC - 1)
        def _finish_cond():
            bin_ref[...] = _tie_break(keys, t_ref[0]).reshape(1, 1, _H)
            cond = jax.nn.relu((acc_ref[...] + bc_ref[0]) * emb_ref[...])
            act_ref[...] = cond
            acc_ref[...] = jnp.zeros((1, _G), jnp.float32)


def _const_spec(shape):
    nd = len(shape)
    return pl.BlockSpec(shape, lambda s: (0,) * nd)


@jax.jit
def kernel(x, embedding_input, W_ih, W_hh, b_lstm, em_W0, em_b0, em_W1, em_b1,
           Wg0, bg0, Wc1, bc1, Wg1, bg1, Wc2, bc2, Wg2, bg2, Wcl, bcl, Wgl, bgl):
    row = lambda v: v.reshape(1, -1)
    wg_cat = jnp.concatenate([Wg1, Wg2, Wgl], axis=1)
    wc_cat = jnp.concatenate([Wc2, Wcl], axis=0)
    bg_cat = jnp.stack([bg0, bg1, bg2, bgl], axis=0).reshape(4, 1, _H)
    bc_cat = jnp.stack([bc1, bc2, bcl], axis=0).reshape(3, 1, _G)

    def wg_im(s):
        p, c = s // _NC, s % _NC
        use = 4 * (p - 2) + c
        hold = 4 * (p - 1)
        return (0, jnp.where(p % 2 == 0, jnp.maximum(use, 0), hold))

    def wc_im(s):
        p, c = s // _NC, s % _NC
        idx = jnp.where(p == 3, c,
              jnp.where(p == 5, 8 + c,
              jnp.where(p < 3, 0, jnp.where(p == 4, 8, 15))))
        return (idx, 0)

    def bg_im(s):
        p = s // _NC
        return (jnp.minimum((p + 1) // 2, 3), 0, 0)

    def bc_im(s):
        p = s // _NC
        return (jnp.minimum(p // 2, 2), 0, 0)

    def out_im(s):
        return (s // _NC // 2, 0, 0)

    in_specs = [
        _const_spec((_SEQ, 64)),
        _const_spec((1, 256)),
        _const_spec((64, 4 * _G)),
        _const_spec((_G, 4 * _G)),
        _const_spec((1, 4 * _G)),
        _const_spec((256, _G)),
        _const_spec((1, _G)),
        _const_spec((_G, _G)),
        _const_spec((1, _G)),
        pl.BlockSpec(memory_space=pl.ANY),
        pl.BlockSpec(memory_space=pl.ANY),
        pl.BlockSpec((_G, _CB), wg_im),
        pl.BlockSpec((_CB, _G), wc_im),
        pl.BlockSpec((1, 1, _H), bg_im),
        pl.BlockSpec((1, 1, _G), bc_im),
    ]
    out_specs = [
        pl.BlockSpec((1, 1, _H), out_im),
        pl.BlockSpec((1, 1, _H), out_im),
    ]

    masks, bins = pl.pallas_call(
        _body,
        grid=(7 * _NC,),
        in_specs=in_specs,
        out_specs=out_specs,
        out_shape=[jax.ShapeDtypeStruct((4, 1, _H), jnp.float32)] * 2,
        scratch_shapes=[
            pltpu.VMEM((_SEQ, 4 * _G), jnp.float32),
            pltpu.VMEM((1, _G), jnp.float32),
            pltpu.VMEM((1, _G), jnp.float32),
            pltpu.VMEM((1, _H), jnp.float32),
            pltpu.VMEM((1, _H), jnp.float32),
            pltpu.VMEM((1, _G), jnp.float32),
            pltpu.VMEM((_G, _H), jnp.float32),
            pltpu.VMEM((_H, _G), jnp.float32),
            pltpu.SMEM((1,), jnp.uint32),
            pltpu.SemaphoreType.DMA,
            pltpu.SemaphoreType.DMA,
        ],
    )(x, row(embedding_input), W_ih.T, W_hh.T, row(b_lstm),
      em_W0, row(em_b0), em_W1, row(em_b1),
      Wg0, Wc1, wg_cat, wc_cat, bg_cat, bc_cat)

    masks = masks.reshape(4, _H)
    bins = bins.reshape(4, _H)
    return (masks[0], masks[1], masks[2], masks[3],
            bins[0], bins[1], bins[2], bins[3])


# speedup vs baseline: 1.0027x; 1.0027x over previous
import functools

import jax
import jax.numpy as jnp
from jax import lax
from jax.experimental import pallas as pl
from jax.experimental.pallas import tpu as pltpu

_G = 512
_H = 8192
_SEQ = 200
_K = 4096
_CB = 1024
_NC = _H // _CB

_dot = functools.partial(jnp.dot, preferred_element_type=jnp.float32)


def _keys_of(raw):
    return lax.bitcast_convert_type(raw, jnp.uint32)


def _vsteps(keys, t, g0, n):
    for j in range(n):
        b = (jnp.uint32(30) - (g0 + j).astype(jnp.uint32))
        cand = t | (jnp.uint32(1) << b)
        cnt = jnp.sum((keys >= cand).astype(jnp.int32))
        t = lax.select(cnt >= _K, cand, t)
    return t


def _tie_break(keys, t):
    cgt = jnp.sum((keys > t).astype(jnp.int32))
    r = _K - cgt
    eq = keys == t
    idx = lax.broadcasted_iota(jnp.int32, (1, _H), 1)

    def istep(i, q):
        b = 12 - i
        cand = q | (jnp.int32(1) << b)
        f = jnp.sum((eq & (idx < cand)).astype(jnp.int32))
        return lax.select(f < r, cand, q)

    q = lax.fori_loop(0, 13, istep, jnp.int32(0))
    member = (keys > t) | (eq & (idx <= q))
    return jnp.where(member & (keys > jnp.uint32(0)),
                     jnp.float32(1.0), jnp.float32(0.0))


def _topk_binary(raw):
    keys = _keys_of(raw)
    t = _vsteps(keys, jnp.uint32(0), jnp.int32(0), 31)
    return _tie_break(keys, t)


def _body(x_ref, ei_ref, wihT_ref, whhT_ref, bl_ref,
          w0_ref, b0_ref, w1_ref, b1_ref,
          wg0_hbm, wc1_hbm, wg_ref, wc_ref, bg_ref, bc_ref,
          mask_ref, bin_ref,
          xw_ref, emb_ref, act_ref, y_ref, raw_ref, acc_ref,
          wg0_v, wc1_v, whhhi_ref, whhlo_ref, t_ref, sem_g, sem_c):
    s = pl.program_id(0)
    p = s // _NC
    c = s % _NC

    @pl.when(s == 0)
    def _prefetch_and_lstm():
        pltpu.make_async_copy(wg0_hbm, wg0_v, sem_g).start()
        pltpu.make_async_copy(wc1_hbm, wc1_v, sem_c).start()

        xw_ref[...] = _dot(x_ref[...], wihT_ref[...])

        w = whhT_ref[...]

        def step(tt, hc):
            h, cc = hc
            gates = xw_ref[pl.ds(tt, 1), :] + _dot(h, w) + bl_ref[...]
            ig = jax.nn.sigmoid(gates[:, 0:_G])
            fg = jax.nn.sigmoid(gates[:, _G:2 * _G])
            gg = jnp.tanh(gates[:, 2 * _G:3 * _G])
            og = jax.nn.sigmoid(gates[:, 3 * _G:4 * _G])
            cc = fg * cc + ig * gg
            h = og * jnp.tanh(cc)
            return (h, cc)

        z = jnp.zeros((1, _G), jnp.float32)
        h, _ = lax.fori_loop(0, _SEQ, step, (z, z))

        emb = jax.nn.relu(_dot(ei_ref[...], w0_ref[...]) + b0_ref[...])
        emb = _dot(emb, w1_ref[...]) + b1_ref[...]
        embedding = emb * h
        emb_ref[...] = embedding
        act_ref[...] = jax.nn.relu(embedding)
        acc_ref[...] = jnp.zeros((1, _G), jnp.float32)
        pltpu.make_async_copy(wg0_hbm, wg0_v, sem_g).wait()

    @pl.when((s == _NC) & (c == 0))
    def _wait_wc1():
        pltpu.make_async_copy(wc1_hbm, wc1_v, sem_c).wait()

    @pl.when(p % 2 == 0)
    def _wg_phase():
        off = pl.multiple_of(c * _CB, _CB)

        def compute(wgm):
            y_c = _dot(act_ref[...], wgm) + bg_ref[0, pl.ds(0, 1), pl.ds(off, _CB)]
            y_ref[pl.ds(0, 1), pl.ds(off, _CB)] = y_c

        @pl.when(p == 0)
        def _use_prefetched():
            compute(wg0_v[:, pl.ds(off, _CB)])

        @pl.when(p > 0)
        def _use_stream():
            compute(wg_ref[...])

        @pl.when(c == _NC - 1)
        def _finish_layer():
            y = y_ref[...]
            mn = jnp.min(y)
            mx = jnp.max(y)
            raw = (y - mn) / (mx - mn)
            raw_ref[...] = raw
            mask_ref[...] = raw.reshape(1, 1, _H)

            @pl.when(p == 6)
            def _last_binary():
                bin_ref[...] = _topk_binary(raw).reshape(1, 1, _H)

    @pl.when(p % 2 == 1)
    def _wc_phase():
        off = pl.multiple_of(c * _CB, _CB)
        raw_c = raw_ref[pl.ds(0, 1), pl.ds(off, _CB)]

        @pl.when(p == 1)
        def _use_prefetched():
            acc_ref[...] += _dot(raw_c, wc1_v[pl.ds(off, _CB), :])

        @pl.when(p > 1)
        def _use_stream():
            acc_ref[...] += _dot(raw_c, wc_ref[...])

        keys = _keys_of(raw_ref[...])

        @pl.when(c == 0)
        def _tk_head():
            t_ref[0] = _vsteps(keys, jnp.uint32(0), jnp.int32(0), 5)

        @pl.when((c >= 1) & (c <= 5))
        def _tk_mid():
            t_ref[0] = _vsteps(keys, t_ref[0], c * 5, 5)

        @pl.when(c == 6)
        def _tk_last():
            t_ref[0] = _vsteps(keys, t_ref[0], jnp.int32(30), 1)

        @pl.when(c == _NC - 1)
        def _finish_cond():
            bin_ref[...] = _tie_break(keys, t_ref[0]).reshape(1, 1, _H)
            cond = jax.nn.relu((acc_ref[...] + bc_ref[0]) * emb_ref[...])
            act_ref[...] = cond
            acc_ref[...] = jnp.zeros((1, _G), jnp.float32)


def _const_spec(shape):
    nd = len(shape)
    return pl.BlockSpec(shape, lambda s: (0,) * nd)


@jax.jit
def kernel(x, embedding_input, W_ih, W_hh, b_lstm, em_W0, em_b0, em_W1, em_b1,
           Wg0, bg0, Wc1, bc1, Wg1, bg1, Wc2, bc2, Wg2, bg2, Wcl, bcl, Wgl, bgl):
    row = lambda v: v.reshape(1, -1)
    wg_cat = jnp.concatenate([Wg1, Wg2, Wgl], axis=1)
    wc_cat = jnp.concatenate([Wc2, Wcl], axis=0)
    bg_cat = jnp.stack([bg0, bg1, bg2, bgl], axis=0).reshape(4, 1, _H)
    bc_cat = jnp.stack([bc1, bc2, bcl], axis=0).reshape(3, 1, _G)

    def wg_im(s):
        p, c = s // _NC, s % _NC
        use = 4 * (p - 2) + c
        hold = 4 * (p - 1)
        return (0, jnp.where(p % 2 == 0, jnp.maximum(use, 0), hold))

    def wc_im(s):
        p, c = s // _NC, s % _NC
        idx = jnp.where(p == 3, c,
              jnp.where(p == 5, 8 + c,
              jnp.where(p < 3, 0, jnp.where(p == 4, 8, 15))))
        return (idx, 0)

    def bg_im(s):
        p = s // _NC
        return (jnp.minimum((p + 1) // 2, 3), 0, 0)

    def bc_im(s):
        p = s // _NC
        return (jnp.minimum(p // 2, 2), 0, 0)

    def out_im(s):
        return (s // _NC // 2, 0, 0)

    in_specs = [
        _const_spec((_SEQ, 64)),
        _const_spec((1, 256)),
        _const_spec((64, 4 * _G)),
        _const_spec((_G, 4 * _G)),
        _const_spec((1, 4 * _G)),
        _const_spec((256, _G)),
        _const_spec((1, _G)),
        _const_spec((_G, _G)),
        _const_spec((1, _G)),
        pl.BlockSpec(memory_space=pl.ANY),
        pl.BlockSpec(memory_space=pl.ANY),
        pl.BlockSpec((_G, _CB), wg_im),
        pl.BlockSpec((_CB, _G), wc_im),
        pl.BlockSpec((1, 1, _H), bg_im),
        pl.BlockSpec((1, 1, _G), bc_im),
    ]
    out_specs = [
        pl.BlockSpec((1, 1, _H), out_im),
        pl.BlockSpec((1, 1, _H), out_im),
    ]

    masks, bins = pl.pallas_call(
        _body,
        grid=(7 * _NC,),
        in_specs=in_specs,
        out_specs=out_specs,
        out_shape=[jax.ShapeDtypeStruct((4, 1, _H), jnp.float32)] * 2,
        scratch_shapes=[
            pltpu.VMEM((_SEQ, 4 * _G), jnp.float32),
            pltpu.VMEM((1, _G), jnp.float32),
            pltpu.VMEM((1, _G), jnp.float32),
            pltpu.VMEM((1, _H), jnp.float32),
            pltpu.VMEM((1, _H), jnp.float32),
            pltpu.VMEM((1, _G), jnp.float32),
            pltpu.VMEM((_G, _H), jnp.float32),
            pltpu.VMEM((_H, _G), jnp.float32),
            pltpu.VMEM((_G, 4 * _G), jnp.bfloat16),
            pltpu.VMEM((_G, 4 * _G), jnp.bfloat16),
            pltpu.SMEM((1,), jnp.uint32),
            pltpu.SemaphoreType.DMA,
            pltpu.SemaphoreType.DMA,
        ],
    )(x, row(embedding_input), W_ih.T, W_hh.T, row(b_lstm),
      em_W0, row(em_b0), em_W1, row(em_b1),
      Wg0, Wc1, wg_cat, wc_cat, bg_cat, bc_cat)

    masks = masks.reshape(4, _H)
    bins = bins.reshape(4, _H)
    return (masks[0], masks[1], masks[2], masks[3],
            bins[0], bins[1], bins[2], bins[3])


# speedup vs baseline: 1.1824x; 1.1793x over previous
import functools

import jax
import jax.numpy as jnp
from jax import lax
from jax.experimental import pallas as pl
from jax.experimental.pallas import tpu as pltpu

_G = 512
_H = 8192
_SEQ = 200
_K = 4096
_CB = 1024
_NC = _H // _CB

_BPC = -(-31 // _NC)

_dot = functools.partial(jnp.dot, preferred_element_type=jnp.float32)


def _keys_of(raw):
    return lax.bitcast_convert_type(raw, jnp.uint32)


def _vsteps(keys, t, g0, n):
    for j in range(n):
        b = (jnp.uint32(30) - (g0 + j).astype(jnp.uint32))
        cand = t | (jnp.uint32(1) << b)
        cnt = jnp.sum((keys >= cand).astype(jnp.int32))
        t = lax.select(cnt >= _K, cand, t)
    return t


def _tie_break(keys, t):
    cgt = jnp.sum((keys > t).astype(jnp.int32))
    r = _K - cgt
    eq = keys == t
    idx = lax.broadcasted_iota(jnp.int32, (1, _H), 1)

    def istep(i, q):
        b = 12 - i
        cand = q | (jnp.int32(1) << b)
        f = jnp.sum((eq & (idx < cand)).astype(jnp.int32))
        return lax.select(f < r, cand, q)

    q = lax.fori_loop(0, 13, istep, jnp.int32(0))
    member = (keys > t) | (eq & (idx <= q))
    return jnp.where(member & (keys > jnp.uint32(0)),
                     jnp.float32(1.0), jnp.float32(0.0))


def _topk_binary(raw):
    keys = _keys_of(raw)
    t = _vsteps(keys, jnp.uint32(0), jnp.int32(0), 31)
    return _tie_break(keys, t)


def _body(x_ref, ei_ref, wihT_ref, whhT_ref, bl_ref,
          w0_ref, b0_ref, w1_ref, b1_ref,
          wg0_hbm, wc1_hbm, wg2_hbm, wcl_hbm, wg_ref, wc_ref, bg_ref, bc_ref,
          mask_ref, bin_ref,
          xw_ref, emb_ref, act_ref, y_ref, raw_ref, acc_ref,
          bufa_v, bufb_v, t_ref, sem_g, sem_c):
    s = pl.program_id(0)
    p = s // _NC
    c = s % _NC

    @pl.when(s == 0)
    def _prefetch_and_lstm():
        pltpu.make_async_copy(wg0_hbm, bufa_v, sem_g).start()
        pltpu.make_async_copy(wc1_hbm, bufb_v, sem_c).start()

        xw_ref[...] = _dot(x_ref[...], wihT_ref[...])

        w = whhT_ref[...]

        def step(tt, hc):
            h, cc = hc
            gates = xw_ref[pl.ds(tt, 1), :] + _dot(h, w) + bl_ref[...]
            ig = jax.nn.sigmoid(gates[:, 0:_G])
            fg = jax.nn.sigmoid(gates[:, _G:2 * _G])
            gg = jnp.tanh(gates[:, 2 * _G:3 * _G])
            og = jax.nn.sigmoid(gates[:, 3 * _G:4 * _G])
            cc = fg * cc + ig * gg
            h = og * jnp.tanh(cc)
            return (h, cc)

        z = jnp.zeros((1, _G), jnp.float32)
        h, _ = lax.fori_loop(0, _SEQ, step, (z, z))

        emb = jax.nn.relu(_dot(ei_ref[...], w0_ref[...]) + b0_ref[...])
        emb = _dot(emb, w1_ref[...]) + b1_ref[...]
        embedding = emb * h
        emb_ref[...] = embedding
        act_ref[...] = jax.nn.relu(embedding)
        acc_ref[...] = jnp.zeros((1, _G), jnp.float32)
        pltpu.make_async_copy(wg0_hbm, bufa_v, sem_g).wait()

    @pl.when(s == _NC)
    def _wait_wc1():
        pltpu.make_async_copy(wc1_hbm, bufb_v, sem_c).wait()

    @pl.when(p == 1)
    def _start_wg2_chunk():
        off = pl.multiple_of(c * _CB, _CB)
        pltpu.make_async_copy(wg2_hbm.at[:, pl.ds(off, _CB)],
                              bufa_v.at[:, pl.ds(off, _CB)], sem_g).start()

    @pl.when(p == 2)
    def _start_wcl_chunk():
        off = pl.multiple_of(c * _CB, _CB)
        pltpu.make_async_copy(wcl_hbm.at[pl.ds(off, _CB), :],
                              bufb_v.at[pl.ds(off, _CB), :], sem_c).start()

    @pl.when(p == 4)
    def _wait_wg2_chunk():
        off = pl.multiple_of(c * _CB, _CB)
        pltpu.make_async_copy(wg2_hbm.at[:, pl.ds(off, _CB)],
                              bufa_v.at[:, pl.ds(off, _CB)], sem_g).wait()

    @pl.when(p == 5)
    def _wait_wcl_chunk():
        off = pl.multiple_of(c * _CB, _CB)
        pltpu.make_async_copy(wcl_hbm.at[pl.ds(off, _CB), :],
                              bufb_v.at[pl.ds(off, _CB), :], sem_c).wait()

    @pl.when(p % 2 == 0)
    def _wg_phase():
        off = pl.multiple_of(c * _CB, _CB)

        def compute(wgm):
            y_c = _dot(act_ref[...], wgm) + bg_ref[0, pl.ds(0, 1), pl.ds(off, _CB)]
            y_ref[pl.ds(0, 1), pl.ds(off, _CB)] = y_c

        @pl.when((p == 0) | (p == 4))
        def _use_prefetched():
            compute(bufa_v[:, pl.ds(off, _CB)])

        @pl.when((p == 2) | (p == 6))
        def _use_stream():
            compute(wg_ref[...])

        @pl.when(c == _NC - 1)
        def _finish_layer():
            y = y_ref[...]
            mn = jnp.min(y)
            mx = jnp.max(y)
            raw = (y - mn) / (mx - mn)
            raw_ref[...] = raw
            mask_ref[...] = raw.reshape(1, 1, _H)

            @pl.when(p == 6)
            def _last_binary():
                bin_ref[...] = _topk_binary(raw).reshape(1, 1, _H)

    @pl.when(p % 2 == 1)
    def _wc_phase():
        off = pl.multiple_of(c * _CB, _CB)
        raw_c = raw_ref[pl.ds(0, 1), pl.ds(off, _CB)]

        @pl.when((p == 1) | (p == 5))
        def _use_prefetched():
            acc_ref[...] += _dot(raw_c, bufb_v[pl.ds(off, _CB), :])

        @pl.when(p == 3)
        def _use_stream():
            acc_ref[...] += _dot(raw_c, wc_ref[...])

        keys = _keys_of(raw_ref[...])

        @pl.when(c < _NC - 1)
        def _tk_mid():
            t0 = lax.select(c == 0, jnp.uint32(0), t_ref[0])
            t_ref[0] = _vsteps(keys, t0, c * _BPC, _BPC)

        @pl.when(c == _NC - 1)
        def _finish_cond():
            t = _vsteps(keys, t_ref[0], jnp.int32((_NC - 1) * _BPC),
                        31 - (_NC - 1) * _BPC)
            bin_ref[...] = _tie_break(keys, t).reshape(1, 1, _H)
            cond = jax.nn.relu((acc_ref[...] + bc_ref[0]) * emb_ref[...])
            act_ref[...] = cond
            acc_ref[...] = jnp.zeros((1, _G), jnp.float32)


def _const_spec(shape):
    nd = len(shape)
    return pl.BlockSpec(shape, lambda s: (0,) * nd)


@jax.jit
def kernel(x, embedding_input, W_ih, W_hh, b_lstm, em_W0, em_b0, em_W1, em_b1,
           Wg0, bg0, Wc1, bc1, Wg1, bg1, Wc2, bc2, Wg2, bg2, Wcl, bcl, Wgl, bgl):
    row = lambda v: v.reshape(1, -1)
    wg_cat = jnp.concatenate([Wg1, Wgl], axis=1)
    wc_cat = Wc2
    bg_cat = jnp.stack([bg0, bg1, bg2, bgl], axis=0).reshape(4, 1, _H)
    bc_cat = jnp.stack([bc1, bc2, bcl], axis=0).reshape(3, 1, _G)

    def wg_im(s):
        p, c = s // _NC, s % _NC
        idx = jnp.where(p == 2, c,
              jnp.where(p == 6, _NC + c,
              jnp.where(p < 2, 0, _NC)))
        return (0, idx)

    def wc_im(s):
        p, c = s // _NC, s % _NC
        idx = jnp.where(p == 3, c, jnp.where(p < 3, 0, _NC - 1))
        return (idx, 0)

    def bg_im(s):
        p = s // _NC
        return (jnp.minimum((p + 1) // 2, 3), 0, 0)

    def bc_im(s):
        p = s // _NC
        return (jnp.minimum(p // 2, 2), 0, 0)

    def out_im(s):
        return (s // _NC // 2, 0, 0)

    in_specs = [
        _const_spec((_SEQ, 64)),
        _const_spec((1, 256)),
        _const_spec((64, 4 * _G)),
        _const_spec((_G, 4 * _G)),
        _const_spec((1, 4 * _G)),
        _const_spec((256, _G)),
        _const_spec((1, _G)),
        _const_spec((_G, _G)),
        _const_spec((1, _G)),
        pl.BlockSpec(memory_space=pl.ANY),
        pl.BlockSpec(memory_space=pl.ANY),
        pl.BlockSpec(memory_space=pl.ANY),
        pl.BlockSpec(memory_space=pl.ANY),
        pl.BlockSpec((_G, _CB), wg_im),
        pl.BlockSpec((_CB, _G), wc_im),
        pl.BlockSpec((1, 1, _H), bg_im),
        pl.BlockSpec((1, 1, _G), bc_im),
    ]
    out_specs = [
        pl.BlockSpec((1, 1, _H), out_im),
        pl.BlockSpec((1, 1, _H), out_im),
    ]

    masks, bins = pl.pallas_call(
        _body,
        grid=(7 * _NC,),
        in_specs=in_specs,
        out_specs=out_specs,
        out_shape=[jax.ShapeDtypeStruct((4, 1, _H), jnp.float32)] * 2,
        scratch_shapes=[
            pltpu.VMEM((_SEQ, 4 * _G), jnp.float32),
            pltpu.VMEM((1, _G), jnp.float32),
            pltpu.VMEM((1, _G), jnp.float32),
            pltpu.VMEM((1, _H), jnp.float32),
            pltpu.VMEM((1, _H), jnp.float32),
            pltpu.VMEM((1, _G), jnp.float32),
            pltpu.VMEM((_G, _H), jnp.float32),
            pltpu.VMEM((_H, _G), jnp.float32),
            pltpu.SMEM((1,), jnp.uint32),
            pltpu.SemaphoreType.DMA,
            pltpu.SemaphoreType.DMA,
        ],
    )(x, row(embedding_input), W_ih.T, W_hh.T, row(b_lstm),
      em_W0, row(em_b0), em_W1, row(em_b1),
      Wg0, Wc1, Wg2, Wcl, wg_cat, wc_cat, bg_cat, bc_cat)

    masks = masks.reshape(4, _H)
    bins = bins.reshape(4, _H)
    return (masks[0], masks[1], masks[2], masks[3],
            bins[0], bins[1], bins[2], bins[3])


# speedup vs baseline: 1.2193x; 1.0312x over previous
import functools

import jax
import jax.numpy as jnp
from jax import lax
from jax.experimental import pallas as pl
from jax.experimental.pallas import tpu as pltpu

_G = 512
_H = 8192
_SEQ = 200
_K = 4096
_CB = 1024
_NC = _H // _CB

_BPC = -(-31 // _NC)
_HC = 3

_dot = functools.partial(jnp.dot, preferred_element_type=jnp.float32)


def _keys_of(raw):
    return lax.bitcast_convert_type(raw, jnp.uint32)


def _vsteps(keys, t, g0, n):
    for j in range(n):
        b = (jnp.uint32(30) - (g0 + j).astype(jnp.uint32))
        cand = t | (jnp.uint32(1) << b)
        cnt = jnp.sum((keys >= cand).astype(jnp.int32))
        t = lax.select(cnt >= _K, cand, t)
    return t


def _tie_break(keys, t):
    cgt = jnp.sum((keys > t).astype(jnp.int32))
    r = _K - cgt
    eq = keys == t
    idx = lax.broadcasted_iota(jnp.int32, (1, _H), 1)

    def istep(i, q):
        b = 12 - i
        cand = q | (jnp.int32(1) << b)
        f = jnp.sum((eq & (idx < cand)).astype(jnp.int32))
        return lax.select(f < r, cand, q)

    q = lax.fori_loop(0, 13, istep, jnp.int32(0))
    member = (keys > t) | (eq & (idx <= q))
    return jnp.where(member & (keys > jnp.uint32(0)),
                     jnp.float32(1.0), jnp.float32(0.0))


def _topk_binary(raw):
    keys = _keys_of(raw)
    t = _vsteps(keys, jnp.uint32(0), jnp.int32(0), 31)
    return _tie_break(keys, t)


def _body(x_ref, ei_ref, wihT_ref, whhT_ref, bl_ref,
          w0_ref, b0_ref, w1_ref, b1_ref,
          wg0_hbm, wc1_hbm, wg2_hbm, wcl_hbm, wgl_hbm, wg_ref, wc_ref, bg_ref, bc_ref,
          mask_ref, bin_ref,
          xw_ref, emb_ref, act_ref, y_ref, raw_ref, acc_ref,
          bufa_v, bufb_v, bufc_v, t_ref, sem_g, sem_c, sem_h):
    s = pl.program_id(0)
    p = s // _NC
    c = s % _NC

    @pl.when(s == 0)
    def _prefetch_and_lstm():
        pltpu.make_async_copy(wg0_hbm, bufa_v, sem_g).start()
        pltpu.make_async_copy(wc1_hbm, bufb_v, sem_c).start()

        xw_ref[...] = _dot(x_ref[...], wihT_ref[...])

        w = whhT_ref[...]

        def step(tt, hc):
            h, cc = hc
            gates = xw_ref[pl.ds(tt, 1), :] + _dot(h, w) + bl_ref[...]
            ig = jax.nn.sigmoid(gates[:, 0:_G])
            fg = jax.nn.sigmoid(gates[:, _G:2 * _G])
            gg = jnp.tanh(gates[:, 2 * _G:3 * _G])
            og = jax.nn.sigmoid(gates[:, 3 * _G:4 * _G])
            cc = fg * cc + ig * gg
            h = og * jnp.tanh(cc)
            return (h, cc)

        z = jnp.zeros((1, _G), jnp.float32)
        h, _ = lax.fori_loop(0, _SEQ, step, (z, z))

        emb = jax.nn.relu(_dot(ei_ref[...], w0_ref[...]) + b0_ref[...])
        emb = _dot(emb, w1_ref[...]) + b1_ref[...]
        embedding = emb * h
        emb_ref[...] = embedding
        act_ref[...] = jax.nn.relu(embedding)
        acc_ref[...] = jnp.zeros((1, _G), jnp.float32)
        pltpu.make_async_copy(wg0_hbm, bufa_v, sem_g).wait()

    @pl.when(s == _NC)
    def _wait_wc1():
        pltpu.make_async_copy(wc1_hbm, bufb_v, sem_c).wait()

    @pl.when(p == 1)
    def _start_wg2_chunk():
        off = pl.multiple_of(c * _CB, _CB)
        pltpu.make_async_copy(wg2_hbm.at[:, pl.ds(off, _CB)],
                              bufa_v.at[:, pl.ds(off, _CB)], sem_g).start()

    @pl.when(p == 2)
    def _start_wcl_chunk():
        off = pl.multiple_of(c * _CB, _CB)
        pltpu.make_async_copy(wcl_hbm.at[pl.ds(off, _CB), :],
                              bufb_v.at[pl.ds(off, _CB), :], sem_c).start()

    @pl.when((p == 3) & (c < _HC))
    def _start_wgl_chunk():
        off = pl.multiple_of(c * _CB, _CB)
        pltpu.make_async_copy(wgl_hbm.at[:, pl.ds(off, _CB)],
                              bufc_v.at[:, pl.ds(off, _CB)], sem_h).start()

    @pl.when((p == 6) & (c < _HC))
    def _wait_wgl_chunk():
        off = pl.multiple_of(c * _CB, _CB)
        pltpu.make_async_copy(wgl_hbm.at[:, pl.ds(off, _CB)],
                              bufc_v.at[:, pl.ds(off, _CB)], sem_h).wait()

    @pl.when(p == 4)
    def _wait_wg2_chunk():
        off = pl.multiple_of(c * _CB, _CB)
        pltpu.make_async_copy(wg2_hbm.at[:, pl.ds(off, _CB)],
                              bufa_v.at[:, pl.ds(off, _CB)], sem_g).wait()

    @pl.when(p == 5)
    def _wait_wcl_chunk():
        off = pl.multiple_of(c * _CB, _CB)
        pltpu.make_async_copy(wcl_hbm.at[pl.ds(off, _CB), :],
                              bufb_v.at[pl.ds(off, _CB), :], sem_c).wait()

    @pl.when(p % 2 == 0)
    def _wg_phase():
        off = pl.multiple_of(c * _CB, _CB)

        def compute(wgm):
            y_c = _dot(act_ref[...], wgm) + bg_ref[0, pl.ds(0, 1), pl.ds(off, _CB)]
            y_ref[pl.ds(0, 1), pl.ds(off, _CB)] = y_c

        @pl.when((p == 0) | (p == 4))
        def _use_prefetched():
            compute(bufa_v[:, pl.ds(off, _CB)])

        @pl.when((p == 6) & (c < _HC))
        def _use_bufc():
            compute(bufc_v[:, pl.ds(off, _CB)])

        @pl.when((p == 2) | ((p == 6) & (c >= _HC)))
        def _use_stream():
            compute(wg_ref[...])

        @pl.when(c == _NC - 1)
        def _finish_layer():
            y = y_ref[...]
            mn = jnp.min(y)
            mx = jnp.max(y)
            raw = (y - mn) / (mx - mn)
            raw_ref[...] = raw
            mask_ref[...] = raw.reshape(1, 1, _H)

            @pl.when(p == 6)
            def _last_binary():
                bin_ref[...] = _topk_binary(raw).reshape(1, 1, _H)

    @pl.when(p % 2 == 1)
    def _wc_phase():
        off = pl.multiple_of(c * _CB, _CB)
        raw_c = raw_ref[pl.ds(0, 1), pl.ds(off, _CB)]

        @pl.when((p == 1) | (p == 5))
        def _use_prefetched():
            acc_ref[...] += _dot(raw_c, bufb_v[pl.ds(off, _CB), :])

        @pl.when(p == 3)
        def _use_stream():
            acc_ref[...] += _dot(raw_c, wc_ref[...])

        keys = _keys_of(raw_ref[...])

        @pl.when(c < _NC - 1)
        def _tk_mid():
            t0 = lax.select(c == 0, jnp.uint32(0), t_ref[0])
            t_ref[0] = _vsteps(keys, t0, c * _BPC, _BPC)

        @pl.when(c == _NC - 1)
        def _finish_cond():
            t = _vsteps(keys, t_ref[0], jnp.int32((_NC - 1) * _BPC),
                        31 - (_NC - 1) * _BPC)
            bin_ref[...] = _tie_break(keys, t).reshape(1, 1, _H)
            cond = jax.nn.relu((acc_ref[...] + bc_ref[0]) * emb_ref[...])
            act_ref[...] = cond
            acc_ref[...] = jnp.zeros((1, _G), jnp.float32)


def _const_spec(shape):
    nd = len(shape)
    return pl.BlockSpec(shape, lambda s: (0,) * nd)


@jax.jit
def kernel(x, embedding_input, W_ih, W_hh, b_lstm, em_W0, em_b0, em_W1, em_b1,
           Wg0, bg0, Wc1, bc1, Wg1, bg1, Wc2, bc2, Wg2, bg2, Wcl, bcl, Wgl, bgl):
    row = lambda v: v.reshape(1, -1)
    wg_cat = jnp.concatenate([Wg1, Wgl[:, _HC * _CB:]], axis=1)
    wc_cat = Wc2
    bg_cat = jnp.stack([bg0, bg1, bg2, bgl], axis=0).reshape(4, 1, _H)
    bc_cat = jnp.stack([bc1, bc2, bcl], axis=0).reshape(3, 1, _G)

    def wg_im(s):
        p, c = s // _NC, s % _NC
        idx = jnp.where(p == 2, c,
              jnp.where(p == 6, jnp.maximum(_NC, _NC + c - _HC),
              jnp.where(p < 2, 0, _NC)))
        return (0, idx)

    def wc_im(s):
        p, c = s // _NC, s % _NC
        idx = jnp.where(p == 3, c, jnp.where(p < 3, 0, _NC - 1))
        return (idx, 0)

    def bg_im(s):
        p = s // _NC
        return (jnp.minimum((p + 1) // 2, 3), 0, 0)

    def bc_im(s):
        p = s // _NC
        return (jnp.minimum(p // 2, 2), 0, 0)

    def out_im(s):
        return (s // _NC // 2, 0, 0)

    in_specs = [
        _const_spec((_SEQ, 64)),
        _const_spec((1, 256)),
        _const_spec((64, 4 * _G)),
        _const_spec((_G, 4 * _G)),
        _const_spec((1, 4 * _G)),
        _const_spec((256, _G)),
        _const_spec((1, _G)),
        _const_spec((_G, _G)),
        _const_spec((1, _G)),
        pl.BlockSpec(memory_space=pl.ANY),
        pl.BlockSpec(memory_space=pl.ANY),
        pl.BlockSpec(memory_space=pl.ANY),
        pl.BlockSpec(memory_space=pl.ANY),
        pl.BlockSpec(memory_space=pl.ANY),
        pl.BlockSpec((_G, _CB), wg_im),
        pl.BlockSpec((_CB, _G), wc_im),
        pl.BlockSpec((1, 1, _H), bg_im),
        pl.BlockSpec((1, 1, _G), bc_im),
    ]
    out_specs = [
        pl.BlockSpec((1, 1, _H), out_im),
        pl.BlockSpec((1, 1, _H), out_im),
    ]

    masks, bins = pl.pallas_call(
        _body,
        grid=(7 * _NC,),
        in_specs=in_specs,
        out_specs=out_specs,
        out_shape=[jax.ShapeDtypeStruct((4, 1, _H), jnp.float32)] * 2,
        scratch_shapes=[
            pltpu.VMEM((_SEQ, 4 * _G), jnp.float32),
            pltpu.VMEM((1, _G), jnp.float32),
            pltpu.VMEM((1, _G), jnp.float32),
            pltpu.VMEM((1, _H), jnp.float32),
            pltpu.VMEM((1, _H), jnp.float32),
            pltpu.VMEM((1, _G), jnp.float32),
            pltpu.VMEM((_G, _H), jnp.float32),
            pltpu.VMEM((_H, _G), jnp.float32),
            pltpu.VMEM((_G, _HC * _CB), jnp.float32),
            pltpu.SMEM((1,), jnp.uint32),
            pltpu.SemaphoreType.DMA,
            pltpu.SemaphoreType.DMA,
            pltpu.SemaphoreType.DMA,
        ],
    )(x, row(embedding_input), W_ih.T, W_hh.T, row(b_lstm),
      em_W0, row(em_b0), em_W1, row(em_b1),
      Wg0, Wc1, Wg2, Wcl, Wgl, wg_cat, wc_cat, bg_cat, bc_cat)

    masks = masks.reshape(4, _H)
    bins = bins.reshape(4, _H)
    return (masks[0], masks[1], masks[2], masks[3],
            bins[0], bins[1], bins[2], bins[3])


# speedup vs baseline: 1.2197x; 1.0003x over previous
import functools

import jax
import jax.numpy as jnp
from jax import lax
from jax.experimental import pallas as pl
from jax.experimental.pallas import tpu as pltpu

_G = 512
_H = 8192
_SEQ = 200
_K = 4096
_CB = 1024
_NC = _H // _CB

_BPC = -(-31 // _NC)
_HC = 3

_dot = functools.partial(jnp.dot, preferred_element_type=jnp.float32)


def _keys_of(raw):
    return lax.bitcast_convert_type(raw, jnp.uint32)


def _vsteps(keys, t, g0, n):
    for j in range(n):
        b = (jnp.uint32(30) - (g0 + j).astype(jnp.uint32))
        cand = t | (jnp.uint32(1) << b)
        cnt = jnp.sum((keys >= cand).astype(jnp.int32))
        t = lax.select(cnt >= _K, cand, t)
    return t


def _tie_break(keys, t):
    cgt = jnp.sum((keys > t).astype(jnp.int32))
    r = _K - cgt
    eq = keys == t
    idx = lax.broadcasted_iota(jnp.int32, (1, _H), 1)

    def istep(i, q):
        b = 12 - i
        cand = q | (jnp.int32(1) << b)
        f = jnp.sum((eq & (idx < cand)).astype(jnp.int32))
        return lax.select(f < r, cand, q)

    q = lax.fori_loop(0, 13, istep, jnp.int32(0))
    member = (keys > t) | (eq & (idx <= q))
    return jnp.where(member & (keys > jnp.uint32(0)),
                     jnp.float32(1.0), jnp.float32(0.0))


def _topk_binary(raw):
    keys = _keys_of(raw)
    t = _vsteps(keys, jnp.uint32(0), jnp.int32(0), 31)
    return _tie_break(keys, t)


def _body(x_ref, ei_ref, wihT_ref, whhT_ref, bl_ref,
          w0_ref, b0_ref, w1_ref, b1_ref,
          wg0_hbm, wc1_hbm, wg2_hbm, wcl_hbm, wgl_hbm, wg1_hbm, wg_ref, wc_ref, bg_ref, bc_ref,
          mask_ref, bin_ref,
          xw_ref, emb_ref, act_ref, y_ref, raw_ref, acc_ref,
          bufa_v, bufb_v, bufc_v, t_ref, sem_g, sem_c, sem_h):
    s = pl.program_id(0)
    p = s // _NC
    c = s % _NC

    @pl.when(s == 0)
    def _prefetch_and_lstm():
        pltpu.make_async_copy(wg0_hbm, bufa_v, sem_g).start()
        pltpu.make_async_copy(wc1_hbm, bufb_v, sem_c).start()
        pltpu.make_async_copy(wg1_hbm.at[:, pl.ds(0, _HC * _CB)], bufc_v,
                              sem_h).start()

        xw_ref[...] = _dot(x_ref[...], wihT_ref[...])

        w = whhT_ref[...]

        def step(tt, hc):
            h, cc = hc
            gates = xw_ref[pl.ds(tt, 1), :] + _dot(h, w) + bl_ref[...]
            ig = jax.nn.sigmoid(gates[:, 0:_G])
            fg = jax.nn.sigmoid(gates[:, _G:2 * _G])
            gg = jnp.tanh(gates[:, 2 * _G:3 * _G])
            og = jax.nn.sigmoid(gates[:, 3 * _G:4 * _G])
            cc = fg * cc + ig * gg
            h = og * jnp.tanh(cc)
            return (h, cc)

        z = jnp.zeros((1, _G), jnp.float32)
        h, _ = lax.fori_loop(0, _SEQ, step, (z, z))

        emb = jax.nn.relu(_dot(ei_ref[...], w0_ref[...]) + b0_ref[...])
        emb = _dot(emb, w1_ref[...]) + b1_ref[...]
        embedding = emb * h
        emb_ref[...] = embedding
        act_ref[...] = jax.nn.relu(embedding)
        acc_ref[...] = jnp.zeros((1, _G), jnp.float32)
        pltpu.make_async_copy(wg0_hbm, bufa_v, sem_g).wait()

    @pl.when(s == _NC)
    def _wait_wc1():
        pltpu.make_async_copy(wc1_hbm, bufb_v, sem_c).wait()

    @pl.when(p == 1)
    def _start_wg2_chunk():
        off = pl.multiple_of(c * _CB, _CB)
        pltpu.make_async_copy(wg2_hbm.at[:, pl.ds(off, _CB)],
                              bufa_v.at[:, pl.ds(off, _CB)], sem_g).start()

    @pl.when(p == 2)
    def _start_wcl_chunk():
        off = pl.multiple_of(c * _CB, _CB)
        pltpu.make_async_copy(wcl_hbm.at[pl.ds(off, _CB), :],
                              bufb_v.at[pl.ds(off, _CB), :], sem_c).start()

    @pl.when((p == 2) & (s % _NC == 0))
    def _wait_wg1_head():
        pltpu.make_async_copy(wg1_hbm.at[:, pl.ds(0, _HC * _CB)], bufc_v,
                              sem_h).wait()

    @pl.when((p == 3) & (c < _HC))
    def _start_wgl_chunk():
        off = pl.multiple_of(c * _CB, _CB)
        pltpu.make_async_copy(wgl_hbm.at[:, pl.ds(off, _CB)],
                              bufc_v.at[:, pl.ds(off, _CB)], sem_h).start()

    @pl.when((p == 6) & (c < _HC))
    def _wait_wgl_chunk():
        off = pl.multiple_of(c * _CB, _CB)
        pltpu.make_async_copy(wgl_hbm.at[:, pl.ds(off, _CB)],
                              bufc_v.at[:, pl.ds(off, _CB)], sem_h).wait()

    @pl.when(p == 4)
    def _wait_wg2_chunk():
        off = pl.multiple_of(c * _CB, _CB)
        pltpu.make_async_copy(wg2_hbm.at[:, pl.ds(off, _CB)],
                              bufa_v.at[:, pl.ds(off, _CB)], sem_g).wait()

    @pl.when(p == 5)
    def _wait_wcl_chunk():
        off = pl.multiple_of(c * _CB, _CB)
        pltpu.make_async_copy(wcl_hbm.at[pl.ds(off, _CB), :],
                              bufb_v.at[pl.ds(off, _CB), :], sem_c).wait()

    @pl.when(p % 2 == 0)
    def _wg_phase():
        off = pl.multiple_of(c * _CB, _CB)

        def compute(wgm):
            y_c = _dot(act_ref[...], wgm) + bg_ref[0, pl.ds(0, 1), pl.ds(off, _CB)]
            y_ref[pl.ds(0, 1), pl.ds(off, _CB)] = y_c

        @pl.when((p == 0) | (p == 4))
        def _use_prefetched():
            compute(bufa_v[:, pl.ds(off, _CB)])

        @pl.when(((p == 2) | (p == 6)) & (c < _HC))
        def _use_bufc():
            compute(bufc_v[:, pl.ds(off, _CB)])

        @pl.when(((p == 2) | (p == 6)) & (c >= _HC))
        def _use_stream():
            compute(wg_ref[...])

        @pl.when(c == _NC - 1)
        def _finish_layer():
            y = y_ref[...]
            mn = jnp.min(y)
            mx = jnp.max(y)
            raw = (y - mn) / (mx - mn)
            raw_ref[...] = raw
            mask_ref[...] = raw.reshape(1, 1, _H)

            @pl.when(p == 6)
            def _last_binary():
                bin_ref[...] = _topk_binary(raw).reshape(1, 1, _H)

    @pl.when(p % 2 == 1)
    def _wc_phase():
        off = pl.multiple_of(c * _CB, _CB)
        raw_c = raw_ref[pl.ds(0, 1), pl.ds(off, _CB)]

        @pl.when((p == 1) | (p == 5))
        def _use_prefetched():
            acc_ref[...] += _dot(raw_c, bufb_v[pl.ds(off, _CB), :])

        @pl.when(p == 3)
        def _use_stream():
            acc_ref[...] += _dot(raw_c, wc_ref[...])

        keys = _keys_of(raw_ref[...])

        @pl.when(c < _NC - 1)
        def _tk_mid():
            t0 = lax.select(c == 0, jnp.uint32(0), t_ref[0])
            t_ref[0] = _vsteps(keys, t0, c * _BPC, _BPC)

        @pl.when(c == _NC - 1)
        def _finish_cond():
            t = _vsteps(keys, t_ref[0], jnp.int32((_NC - 1) * _BPC),
                        31 - (_NC - 1) * _BPC)
            bin_ref[...] = _tie_break(keys, t).reshape(1, 1, _H)
            cond = jax.nn.relu((acc_ref[...] + bc_ref[0]) * emb_ref[...])
            act_ref[...] = cond
            acc_ref[...] = jnp.zeros((1, _G), jnp.float32)


def _const_spec(shape):
    nd = len(shape)
    return pl.BlockSpec(shape, lambda s: (0,) * nd)


@jax.jit
def kernel(x, embedding_input, W_ih, W_hh, b_lstm, em_W0, em_b0, em_W1, em_b1,
           Wg0, bg0, Wc1, bc1, Wg1, bg1, Wc2, bc2, Wg2, bg2, Wcl, bcl, Wgl, bgl):
    row = lambda v: v.reshape(1, -1)
    wg_cat = jnp.concatenate([Wg1[:, _HC * _CB:], Wgl[:, _HC * _CB:]], axis=1)
    wc_cat = Wc2
    bg_cat = jnp.stack([bg0, bg1, bg2, bgl], axis=0).reshape(4, 1, _H)
    bc_cat = jnp.stack([bc1, bc2, bcl], axis=0).reshape(3, 1, _G)

    _NT = _NC - _HC

    def wg_im(s):
        p, c = s // _NC, s % _NC
        idx = jnp.where(p == 2, jnp.maximum(c - _HC, 0),
              jnp.where(p == 6, _NT + jnp.maximum(c - _HC, 0),
              jnp.where(p < 2, 0, _NT)))
        return (0, idx)

    def wc_im(s):
        p, c = s // _NC, s % _NC
        idx = jnp.where(p == 3, c, jnp.where(p < 3, 0, _NC - 1))
        return (idx, 0)

    def bg_im(s):
        p = s // _NC
        return (jnp.minimum((p + 1) // 2, 3), 0, 0)

    def bc_im(s):
        p = s // _NC
        return (jnp.minimum(p // 2, 2), 0, 0)

    def out_im(s):
        return (s // _NC // 2, 0, 0)

    in_specs = [
        _const_spec((_SEQ, 64)),
        _const_spec((1, 256)),
        _const_spec((64, 4 * _G)),
        _const_spec((_G, 4 * _G)),
        _const_spec((1, 4 * _G)),
        _const_spec((256, _G)),
        _const_spec((1, _G)),
        _const_spec((_G, _G)),
        _const_spec((1, _G)),
        pl.BlockSpec(memory_space=pl.ANY),
        pl.BlockSpec(memory_space=pl.ANY),
        pl.BlockSpec(memory_space=pl.ANY),
        pl.BlockSpec(memory_space=pl.ANY),
        pl.BlockSpec(memory_space=pl.ANY),
        pl.BlockSpec(memory_space=pl.ANY),
        pl.BlockSpec((_G, _CB), wg_im),
        pl.BlockSpec((_CB, _G), wc_im),
        pl.BlockSpec((1, 1, _H), bg_im),
        pl.BlockSpec((1, 1, _G), bc_im),
    ]
    out_specs = [
        pl.BlockSpec((1, 1, _H), out_im),
        pl.BlockSpec((1, 1, _H), out_im),
    ]

    masks, bins = pl.pallas_call(
        _body,
        grid=(7 * _NC,),
        in_specs=in_specs,
        out_specs=out_specs,
        out_shape=[jax.ShapeDtypeStruct((4, 1, _H), jnp.float32)] * 2,
        scratch_shapes=[
            pltpu.VMEM((_SEQ, 4 * _G), jnp.float32),
            pltpu.VMEM((1, _G), jnp.float32),
            pltpu.VMEM((1, _G), jnp.float32),
            pltpu.VMEM((1, _H), jnp.float32),
            pltpu.VMEM((1, _H), jnp.float32),
            pltpu.VMEM((1, _G), jnp.float32),
            pltpu.VMEM((_G, _H), jnp.float32),
            pltpu.VMEM((_H, _G), jnp.float32),
            pltpu.VMEM((_G, _HC * _CB), jnp.float32),
            pltpu.SMEM((1,), jnp.uint32),
            pltpu.SemaphoreType.DMA,
            pltpu.SemaphoreType.DMA,
            pltpu.SemaphoreType.DMA,
        ],
    )(x, row(embedding_input), W_ih.T, W_hh.T, row(b_lstm),
      em_W0, row(em_b0), em_W1, row(em_b1),
      Wg0, Wc1, Wg2, Wcl, Wgl, Wg1, wg_cat, wc_cat, bg_cat, bc_cat)

    masks = masks.reshape(4, _H)
    bins = bins.reshape(4, _H)
    return (masks[0], masks[1], masks[2], masks[3],
            bins[0], bins[1], bins[2], bins[3])


# speedup vs baseline: 1.2566x; 1.0303x over previous
import functools

import jax
import jax.numpy as jnp
from jax import lax
from jax.experimental import pallas as pl
from jax.experimental.pallas import tpu as pltpu

_G = 512
_H = 8192
_SEQ = 200
_K = 4096
_CB = 1024
_NC = _H // _CB

_BPC = -(-31 // _NC)
_HC = 3

_dot = functools.partial(jnp.dot, preferred_element_type=jnp.float32)


def _keys_of(raw):
    return lax.bitcast_convert_type(raw, jnp.uint32)


def _vsteps(keys, t, g0, n):
    for j in range(n):
        b = (jnp.uint32(30) - (g0 + j).astype(jnp.uint32))
        cand = t | (jnp.uint32(1) << b)
        cnt = jnp.sum((keys >= cand).astype(jnp.int32))
        t = lax.select(cnt >= _K, cand, t)
    return t


def _tie_break(keys, t):
    cgt = jnp.sum((keys > t).astype(jnp.int32))
    r = _K - cgt
    eq = keys == t
    idx = lax.broadcasted_iota(jnp.int32, (1, _H), 1)

    def istep(i, q):
        b = 12 - i
        cand = q | (jnp.int32(1) << b)
        f = jnp.sum((eq & (idx < cand)).astype(jnp.int32))
        return lax.select(f < r, cand, q)

    q = lax.fori_loop(0, 13, istep, jnp.int32(0))
    member = (keys > t) | (eq & (idx <= q))
    return jnp.where(member & (keys > jnp.uint32(0)),
                     jnp.float32(1.0), jnp.float32(0.0))


def _topk_binary(raw):
    keys = _keys_of(raw)
    jods = lax.broadcasted_iota(jnp.uint32, (15, 1), 0) + jnp.uint32(1)
    t = jnp.uint32(0)
    for sh, nj in [(27, 15), (23, 15), (19, 15), (15, 15),
                   (11, 15), (7, 15), (3, 15), (0, 7)]:
        jj = jods[:nj]
        cands = t + (jj << jnp.uint32(sh))
        cnt = jnp.sum((keys >= cands).astype(jnp.int32), axis=1)
        v = jnp.sum((cnt >= _K).astype(jnp.int32)).astype(jnp.uint32)
        t = t + (v << jnp.uint32(sh))

    cgt = jnp.sum((keys > t).astype(jnp.int32))
    r = _K - cgt
    eq = keys == t
    idx = lax.broadcasted_iota(jnp.int32, (1, _H), 1)
    jodi = lax.broadcasted_iota(jnp.int32, (15, 1), 0) + 1
    q = jnp.int32(0)
    for sh, nj in [(9, 15), (5, 15), (1, 15), (0, 1)]:
        jj = jodi[:nj]
        cands = q + (jj << sh)
        f = jnp.sum((eq & (idx < cands)).astype(jnp.int32), axis=1)
        v = jnp.sum((f < r).astype(jnp.int32))
        q = q + (v << sh)

    member = (keys > t) | (eq & (idx <= q))
    return jnp.where(member & (keys > jnp.uint32(0)),
                     jnp.float32(1.0), jnp.float32(0.0))


def _body(x_ref, ei_ref, wihT_ref, whhT_ref, bl_ref,
          w0_ref, b0_ref, w1_ref, b1_ref,
          wg0_hbm, wc1_hbm, wg2_hbm, wcl_hbm, wgl_hbm, wg1_hbm, wg_ref, wc_ref, bg_ref, bc_ref,
          mask_ref, bin_ref,
          xw_ref, emb_ref, act_ref, y_ref, raw_ref, acc_ref,
          bufa_v, bufb_v, bufc_v, t_ref, sem_g, sem_c, sem_h):
    s = pl.program_id(0)
    p = s // _NC
    c = s % _NC

    @pl.when(s == 0)
    def _prefetch_and_lstm():
        pltpu.make_async_copy(wg0_hbm, bufa_v, sem_g).start()
        pltpu.make_async_copy(wc1_hbm, bufb_v, sem_c).start()
        pltpu.make_async_copy(wg1_hbm.at[:, pl.ds(0, _HC * _CB)], bufc_v,
                              sem_h).start()

        xw_ref[...] = _dot(x_ref[...], wihT_ref[...])

        w = whhT_ref[...]

        def step(tt, hc):
            h, cc = hc
            gates = xw_ref[pl.ds(tt, 1), :] + _dot(h, w) + bl_ref[...]
            ig = jax.nn.sigmoid(gates[:, 0:_G])
            fg = jax.nn.sigmoid(gates[:, _G:2 * _G])
            gg = jnp.tanh(gates[:, 2 * _G:3 * _G])
            og = jax.nn.sigmoid(gates[:, 3 * _G:4 * _G])
            cc = fg * cc + ig * gg
            h = og * jnp.tanh(cc)
            return (h, cc)

        z = jnp.zeros((1, _G), jnp.float32)
        h, _ = lax.fori_loop(0, _SEQ, step, (z, z))

        emb = jax.nn.relu(_dot(ei_ref[...], w0_ref[...]) + b0_ref[...])
        emb = _dot(emb, w1_ref[...]) + b1_ref[...]
        embedding = emb * h
        emb_ref[...] = embedding
        act_ref[...] = jax.nn.relu(embedding)
        acc_ref[...] = jnp.zeros((1, _G), jnp.float32)
        pltpu.make_async_copy(wg0_hbm, bufa_v, sem_g).wait()

    @pl.when(s == _NC)
    def _wait_wc1():
        pltpu.make_async_copy(wc1_hbm, bufb_v, sem_c).wait()

    @pl.when(p == 1)
    def _start_wg2_chunk():
        off = pl.multiple_of(c * _CB, _CB)
        pltpu.make_async_copy(wg2_hbm.at[:, pl.ds(off, _CB)],
                              bufa_v.at[:, pl.ds(off, _CB)], sem_g).start()

    @pl.when(p == 2)
    def _start_wcl_chunk():
        off = pl.multiple_of(c * _CB, _CB)
        pltpu.make_async_copy(wcl_hbm.at[pl.ds(off, _CB), :],
                              bufb_v.at[pl.ds(off, _CB), :], sem_c).start()

    @pl.when((p == 2) & (s % _NC == 0))
    def _wait_wg1_head():
        pltpu.make_async_copy(wg1_hbm.at[:, pl.ds(0, _HC * _CB)], bufc_v,
                              sem_h).wait()

    @pl.when((p == 3) & (c < _HC))
    def _start_wgl_chunk():
        off = pl.multiple_of(c * _CB, _CB)
        pltpu.make_async_copy(wgl_hbm.at[:, pl.ds(off, _CB)],
                              bufc_v.at[:, pl.ds(off, _CB)], sem_h).start()

    @pl.when((p == 6) & (c < _HC))
    def _wait_wgl_chunk():
        off = pl.multiple_of(c * _CB, _CB)
        pltpu.make_async_copy(wgl_hbm.at[:, pl.ds(off, _CB)],
                              bufc_v.at[:, pl.ds(off, _CB)], sem_h).wait()

    @pl.when(p == 4)
    def _wait_wg2_chunk():
        off = pl.multiple_of(c * _CB, _CB)
        pltpu.make_async_copy(wg2_hbm.at[:, pl.ds(off, _CB)],
                              bufa_v.at[:, pl.ds(off, _CB)], sem_g).wait()

    @pl.when(p == 5)
    def _wait_wcl_chunk():
        off = pl.multiple_of(c * _CB, _CB)
        pltpu.make_async_copy(wcl_hbm.at[pl.ds(off, _CB), :],
                              bufb_v.at[pl.ds(off, _CB), :], sem_c).wait()

    @pl.when(p % 2 == 0)
    def _wg_phase():
        off = pl.multiple_of(c * _CB, _CB)

        def compute(wgm):
            y_c = _dot(act_ref[...], wgm) + bg_ref[0, pl.ds(0, 1), pl.ds(off, _CB)]
            y_ref[pl.ds(0, 1), pl.ds(off, _CB)] = y_c

        @pl.when((p == 0) | (p == 4))
        def _use_prefetched():
            compute(bufa_v[:, pl.ds(off, _CB)])

        @pl.when(((p == 2) | (p == 6)) & (c < _HC))
        def _use_bufc():
            compute(bufc_v[:, pl.ds(off, _CB)])

        @pl.when(((p == 2) | (p == 6)) & (c >= _HC))
        def _use_stream():
            compute(wg_ref[...])

        @pl.when(c == _NC - 1)
        def _finish_layer():
            y = y_ref[...]
            mn = jnp.min(y)
            mx = jnp.max(y)
            raw = (y - mn) / (mx - mn)
            raw_ref[...] = raw
            mask_ref[...] = raw.reshape(1, 1, _H)

            @pl.when(p == 6)
            def _last_binary():
                bin_ref[...] = _topk_binary(raw).reshape(1, 1, _H)

    @pl.when(p % 2 == 1)
    def _wc_phase():
        off = pl.multiple_of(c * _CB, _CB)
        raw_c = raw_ref[pl.ds(0, 1), pl.ds(off, _CB)]

        @pl.when((p == 1) | (p == 5))
        def _use_prefetched():
            acc_ref[...] += _dot(raw_c, bufb_v[pl.ds(off, _CB), :])

        @pl.when(p == 3)
        def _use_stream():
            acc_ref[...] += _dot(raw_c, wc_ref[...])

        keys = _keys_of(raw_ref[...])

        @pl.when(c < _NC - 1)
        def _tk_mid():
            t0 = lax.select(c == 0, jnp.uint32(0), t_ref[0])
            t_ref[0] = _vsteps(keys, t0, c * _BPC, _BPC)

        @pl.when(c == _NC - 1)
        def _finish_cond():
            t = _vsteps(keys, t_ref[0], jnp.int32((_NC - 1) * _BPC),
                        31 - (_NC - 1) * _BPC)
            bin_ref[...] = _tie_break(keys, t).reshape(1, 1, _H)
            cond = jax.nn.relu((acc_ref[...] + bc_ref[0]) * emb_ref[...])
            act_ref[...] = cond
            acc_ref[...] = jnp.zeros((1, _G), jnp.float32)


def _const_spec(shape):
    nd = len(shape)
    return pl.BlockSpec(shape, lambda s: (0,) * nd)


@jax.jit
def kernel(x, embedding_input, W_ih, W_hh, b_lstm, em_W0, em_b0, em_W1, em_b1,
           Wg0, bg0, Wc1, bc1, Wg1, bg1, Wc2, bc2, Wg2, bg2, Wcl, bcl, Wgl, bgl):
    row = lambda v: v.reshape(1, -1)
    wg_cat = jnp.concatenate([Wg1[:, _HC * _CB:], Wgl[:, _HC * _CB:]], axis=1)
    wc_cat = Wc2
    bg_cat = jnp.stack([bg0, bg1, bg2, bgl], axis=0).reshape(4, 1, _H)
    bc_cat = jnp.stack([bc1, bc2, bcl], axis=0).reshape(3, 1, _G)

    _NT = _NC - _HC

    def wg_im(s):
        p, c = s // _NC, s % _NC
        idx = jnp.where(p == 2, jnp.maximum(c - _HC, 0),
              jnp.where(p == 6, _NT + jnp.maximum(c - _HC, 0),
              jnp.where(p < 2, 0, _NT)))
        return (0, idx)

    def wc_im(s):
        p, c = s // _NC, s % _NC
        idx = jnp.where(p == 3, c, jnp.where(p < 3, 0, _NC - 1))
        return (idx, 0)

    def bg_im(s):
        p = s // _NC
        return (jnp.minimum((p + 1) // 2, 3), 0, 0)

    def bc_im(s):
        p = s // _NC
        return (jnp.minimum(p // 2, 2), 0, 0)

    def out_im(s):
        return (s // _NC // 2, 0, 0)

    in_specs = [
        _const_spec((_SEQ, 64)),
        _const_spec((1, 256)),
        _const_spec((64, 4 * _G)),
        _const_spec((_G, 4 * _G)),
        _const_spec((1, 4 * _G)),
        _const_spec((256, _G)),
        _const_spec((1, _G)),
        _const_spec((_G, _G)),
        _const_spec((1, _G)),
        pl.BlockSpec(memory_space=pl.ANY),
        pl.BlockSpec(memory_space=pl.ANY),
        pl.BlockSpec(memory_space=pl.ANY),
        pl.BlockSpec(memory_space=pl.ANY),
        pl.BlockSpec(memory_space=pl.ANY),
        pl.BlockSpec(memory_space=pl.ANY),
        pl.BlockSpec((_G, _CB), wg_im),
        pl.BlockSpec((_CB, _G), wc_im),
        pl.BlockSpec((1, 1, _H), bg_im),
        pl.BlockSpec((1, 1, _G), bc_im),
    ]
    out_specs = [
        pl.BlockSpec((1, 1, _H), out_im),
        pl.BlockSpec((1, 1, _H), out_im),
    ]

    masks, bins = pl.pallas_call(
        _body,
        grid=(7 * _NC,),
        in_specs=in_specs,
        out_specs=out_specs,
        out_shape=[jax.ShapeDtypeStruct((4, 1, _H), jnp.float32)] * 2,
        scratch_shapes=[
            pltpu.VMEM((_SEQ, 4 * _G), jnp.float32),
            pltpu.VMEM((1, _G), jnp.float32),
            pltpu.VMEM((1, _G), jnp.float32),
            pltpu.VMEM((1, _H), jnp.float32),
            pltpu.VMEM((1, _H), jnp.float32),
            pltpu.VMEM((1, _G), jnp.float32),
            pltpu.VMEM((_G, _H), jnp.float32),
            pltpu.VMEM((_H, _G), jnp.float32),
            pltpu.VMEM((_G, _HC * _CB), jnp.float32),
            pltpu.SMEM((1,), jnp.uint32),
            pltpu.SemaphoreType.DMA,
            pltpu.SemaphoreType.DMA,
            pltpu.SemaphoreType.DMA,
        ],
    )(x, row(embedding_input), W_ih.T, W_hh.T, row(b_lstm),
      em_W0, row(em_b0), em_W1, row(em_b1),
      Wg0, Wc1, Wg2, Wcl, Wgl, Wg1, wg_cat, wc_cat, bg_cat, bc_cat)

    masks = masks.reshape(4, _H)
    bins = bins.reshape(4, _H)
    return (masks[0], masks[1], masks[2], masks[3],
            bins[0], bins[1], bins[2], bins[3])
